# baseline (speedup 1.0000x reference)
import functools

import jax
import jax.numpy as jnp
from jax import lax
from jax.experimental import pallas as pl
from jax.experimental.pallas import tpu as pltpu
from jax.experimental.pallas import tpu_sc as plsc

N_PAD = 10240
DFEAT = 128
NW = 32
CSZ = 128
NCHUNK = 2560
E_PAD = NCHUNK * CSZ
WCH = NCHUNK // NW
CPP = 40
DEG_CH = NCHUNK // NW
DEG_CPP = 40
TROWS = N_PAD // 16
DUMMY_DST = N_PAD - 1
PBUCK = 128
POOL_W = 16
POOL_ROWS = N_PAD // POOL_W
NUM_G = 64

_MESH = plsc.VectorSubcoreMesh(core_axis_name="c", subcore_axis_name="s")


def _sc_degree(dst3, ones8, zc8):
  @functools.partial(
      pl.kernel,
      out_type=(jax.ShapeDtypeStruct((N_PAD, 8), jnp.float32),
                jax.ShapeDtypeStruct((N_PAD, 8), jnp.float32)),
      mesh=_MESH,
      scratch_types=[
          pltpu.VMEM_SHARED((N_PAD, 8), jnp.float32),
          pltpu.VMEM((DEG_CPP, CSZ), jnp.int32),
          pltpu.VMEM((CSZ, 8), jnp.float32),
          pltpu.SemaphoreType.DMA,
      ],
  )
  def k(dst_hbm, ones_hbm, zc_hbm, outa_hbm, outb_hbm, table, dst_v, ones_v,
        sem):
    c = lax.axis_index("c")
    s = lax.axis_index("s")
    wid = c * 16 + s
    pltpu.sync_copy(zc_hbm, table.at[pl.ds(s * TROWS, TROWS)])
    pltpu.sync_copy(ones_hbm, ones_v)
    plsc.subcore_barrier()

    def fire(j, carry):
      pltpu.async_copy(ones_v, table.at[dst_v.at[j]], sem, add=True)
      return carry

    def drain(j, carry):
      pltpu.make_async_copy(ones_v, table.at[dst_v.at[j]], sem).wait()
      return carry

    for ph in range(DEG_CH // DEG_CPP):
      pltpu.sync_copy(dst_hbm.at[pl.ds(wid * DEG_CH + ph * DEG_CPP, DEG_CPP)],
                      dst_v)
      lax.fori_loop(0, DEG_CPP, fire, 0)
      lax.fori_loop(0, DEG_CPP, drain, 0)
    plsc.subcore_barrier()

    @pl.when(c == 0)
    def _():
      pltpu.sync_copy(table.at[pl.ds(s * TROWS, TROWS)],
                      outa_hbm.at[pl.ds(s * TROWS, TROWS)])

    @pl.when(c == 1)
    def _():
      pltpu.sync_copy(table.at[pl.ds(s * TROWS, TROWS)],
                      outb_hbm.at[pl.ds(s * TROWS, TROWS)])

  return k(dst3, ones8, zc8)


def _sc_scatter(m, src3, dst3):
  @functools.partial(
      pl.kernel,
      out_type=(jax.ShapeDtypeStruct((N_PAD, DFEAT), jnp.float32),
                jax.ShapeDtypeStruct((N_PAD, DFEAT), jnp.float32)),
      mesh=_MESH,
      scratch_types=[
          pltpu.VMEM_SHARED((N_PAD, DFEAT), jnp.float32),
          pltpu.VMEM((CPP, CSZ), jnp.int32),
          pltpu.VMEM((CPP, CSZ), jnp.int32),
          pltpu.VMEM((CSZ, DFEAT), jnp.float32),
          pltpu.VMEM((CSZ, DFEAT), jnp.float32),
          pltpu.SemaphoreType.DMA,
          pltpu.SemaphoreType.DMA,
          pltpu.SemaphoreType.DMA,
          pltpu.SemaphoreType.DMA,
      ],
  )
  def k(m_hbm, src_hbm, dst_hbm, outa_hbm, outb_hbm,
        acc, src_v, dst_v, rows0, rows1, sem0, sem1, sems0, sems1):
    c = lax.axis_index("c")
    s = lax.axis_index("s")
    wid = c * 16 + s

    def zbody(i, carry):
      rows0[i // 8, pl.ds((i % 8) * 16, 16)] = jnp.zeros((16,), jnp.float32)
      return carry

    lax.fori_loop(0, CSZ * 8, zbody, 0)
    for kblk in range(TROWS // CSZ):
      pltpu.sync_copy(rows0, acc.at[pl.ds(s * TROWS + kblk * CSZ, CSZ)])
    plsc.subcore_barrier()

    def body(i, carry):
      j0 = 2 * i
      j1 = j0 + 1
      pltpu.make_async_copy(m_hbm.at[src_v.at[j0]], rows0, sem0).wait()
      pltpu.async_copy(rows0, acc.at[dst_v.at[j0]], sems0, add=True)
      pltpu.make_async_copy(m_hbm.at[src_v.at[j1]], rows1, sem1).wait()
      pltpu.async_copy(rows1, acc.at[dst_v.at[j1]], sems1, add=True)
      pltpu.make_async_copy(rows0, acc.at[dst_v.at[j0]], sems0).wait()

      @pl.when(j0 + 2 < CPP)
      def _():
        pltpu.async_copy(m_hbm.at[src_v.at[j0 + 2]], rows0, sem0)

      pltpu.make_async_copy(rows1, acc.at[dst_v.at[j1]], sems1).wait()

      @pl.when(j1 + 2 < CPP)
      def _():
        pltpu.async_copy(m_hbm.at[src_v.at[j1 + 2]], rows1, sem1)

      return carry

    def phase(ph, carry):
      base = wid * WCH + ph * CPP
      pltpu.sync_copy(src_hbm.at[pl.ds(base, CPP)], src_v)
      pltpu.sync_copy(dst_hbm.at[pl.ds(base, CPP)], dst_v)
      pltpu.async_copy(m_hbm.at[src_v.at[0]], rows0, sem0)
      pltpu.async_copy(m_hbm.at[src_v.at[1]], rows1, sem1)
      lax.fori_loop(0, CPP // 2, body, 0)
      return carry

    lax.fori_loop(0, WCH // CPP, phase, 0)
    plsc.subcore_barrier()

    @pl.when(c == 0)
    def _():
      pltpu.sync_copy(acc.at[pl.ds(s * TROWS, TROWS)],
                      outa_hbm.at[pl.ds(s * TROWS, TROWS)])

    @pl.when(c == 1)
    def _():
      pltpu.sync_copy(acc.at[pl.ds(s * TROWS, TROWS)],
                      outb_hbm.at[pl.ds(s * TROWS, TROWS)])

  return k(m, src3, dst3)


def _sc_pool(z8, batch3, zc8):
  @functools.partial(
      pl.kernel,
      out_type=(jax.ShapeDtypeStruct((PBUCK, 8), jnp.float32),
                jax.ShapeDtypeStruct((PBUCK, 8), jnp.float32)),
      mesh=_MESH,
      scratch_types=[
          pltpu.VMEM_SHARED((PBUCK, 8), jnp.float32),
          pltpu.VMEM((POOL_ROWS // CSZ, CSZ), jnp.int32),
          pltpu.VMEM((POOL_ROWS, 8), jnp.float32),
      ],
  )
  def k(z_hbm, b_hbm, zc_hbm, outa_hbm, outb_hbm, table, b_v, z_v):
    c = lax.axis_index("c")
    s = lax.axis_index("s")

    @pl.when(s == 0)
    def _():
      pltpu.sync_copy(zc_hbm.at[pl.ds(0, PBUCK)], table)

    plsc.subcore_barrier()

    @pl.when(s < POOL_W // 2)
    def _():
      pw = c * (POOL_W // 2) + s
      pltpu.sync_copy(z_hbm.at[pl.ds(pw * POOL_ROWS, POOL_ROWS)], z_v)
      pltpu.sync_copy(b_hbm.at[pw], b_v)

      def body(j, carry):
        pltpu.sync_copy(z_v.at[pl.ds(j * CSZ, CSZ)],
                        table.at[b_v.at[j]], add=True)
        return carry

      lax.fori_loop(0, POOL_ROWS // CSZ, body, 0)

    plsc.subcore_barrier()

    @pl.when((s == 0) & (c == 0))
    def _():
      pltpu.sync_copy(table, outa_hbm)

    @pl.when((s == 0) & (c == 1))
    def _():
      pltpu.sync_copy(table, outb_hbm)

  return k(z8, batch3, zc8)


_BLK = 1024


def _tc_scale_matmul(x, w, dga, dgb):
  def body(x_ref, w_ref, da_ref, db_ref, o_ref):
    dinv = lax.rsqrt(da_ref[:, 0:1] + db_ref[:, 0:1] + 1.0)
    o_ref[...] = dinv * jnp.dot(x_ref[...], w_ref[...],
                                preferred_element_type=jnp.float32,
                                precision=lax.Precision.HIGHEST)

  return pl.pallas_call(
      body,
      grid=(N_PAD // _BLK,),
      in_specs=[
          pl.BlockSpec((_BLK, DFEAT), lambda i: (i, 0)),
          pl.BlockSpec((DFEAT, DFEAT), lambda i: (0, 0)),
          pl.BlockSpec((_BLK, 8), lambda i: (i, 0)),
          pl.BlockSpec((_BLK, 8), lambda i: (i, 0)),
      ],
      out_specs=pl.BlockSpec((_BLK, DFEAT), lambda i: (i, 0)),
      out_shape=jax.ShapeDtypeStruct((N_PAD, DFEAT), jnp.float32),
  )(x, w, dga, dgb)


def _tc_layer2(s1a, s1b, h1, dga, dgb, b1, w2):
  def body(sa_ref, sb_ref, h_ref, da_ref, db_ref, b_ref, w_ref, o_ref):
    dinv = lax.rsqrt(da_ref[:, 0:1] + db_ref[:, 0:1] + 1.0)
    t = dinv * (sa_ref[...] + sb_ref[...] + h_ref[...]) + b_ref[...]
    t = jnp.maximum(t, 0.0)
    o_ref[...] = dinv * jnp.dot(t, w_ref[...],
                                preferred_element_type=jnp.float32,
                                precision=lax.Precision.HIGHEST)

  return pl.pallas_call(
      body,
      grid=(N_PAD // _BLK,),
      in_specs=[
          pl.BlockSpec((_BLK, DFEAT), lambda i: (i, 0)),
          pl.BlockSpec((_BLK, DFEAT), lambda i: (i, 0)),
          pl.BlockSpec((_BLK, DFEAT), lambda i: (i, 0)),
          pl.BlockSpec((_BLK, 8), lambda i: (i, 0)),
          pl.BlockSpec((_BLK, 8), lambda i: (i, 0)),
          pl.BlockSpec((1, DFEAT), lambda i: (0, 0)),
          pl.BlockSpec((DFEAT, DFEAT), lambda i: (0, 0)),
      ],
      out_specs=pl.BlockSpec((_BLK, DFEAT), lambda i: (i, 0)),
      out_shape=jax.ShapeDtypeStruct((N_PAD, DFEAT), jnp.float32),
  )(s1a, s1b, h1, dga, dgb, b1, w2)


def _tc_head(s2a, s2b, h2, dga, dgb, b2, wfc8):
  def body(sa_ref, sb_ref, h_ref, da_ref, db_ref, b_ref, w_ref, o_ref):
    dinv = lax.rsqrt(da_ref[:, 0:1] + db_ref[:, 0:1] + 1.0)
    t = dinv * (sa_ref[...] + sb_ref[...] + h_ref[...]) + b_ref[...]
    z8 = jnp.dot(t, w_ref[...], preferred_element_type=jnp.float32,
                 precision=lax.Precision.HIGHEST)
    lane = lax.broadcasted_iota(jnp.int32, (_BLK, 8), 1)
    o_ref[...] = z8 + (lane == 1).astype(jnp.float32)

  return pl.pallas_call(
      body,
      grid=(N_PAD // _BLK,),
      in_specs=[
          pl.BlockSpec((_BLK, DFEAT), lambda i: (i, 0)),
          pl.BlockSpec((_BLK, DFEAT), lambda i: (i, 0)),
          pl.BlockSpec((_BLK, DFEAT), lambda i: (i, 0)),
          pl.BlockSpec((_BLK, 8), lambda i: (i, 0)),
          pl.BlockSpec((_BLK, 8), lambda i: (i, 0)),
          pl.BlockSpec((1, DFEAT), lambda i: (0, 0)),
          pl.BlockSpec((DFEAT, 8), lambda i: (0, 0)),
      ],
      out_specs=pl.BlockSpec((_BLK, 8), lambda i: (i, 0)),
      out_shape=jax.ShapeDtypeStruct((N_PAD, 8), jnp.float32),
  )(s2a, s2b, h2, dga, dgb, b2, wfc8)


def kernel(x, edge_index, batch, W1, b1, W2, b2, Wfc, bfc):
  n = x.shape[0]
  e = edge_index.shape[1]

  xp = jnp.pad(x.astype(jnp.float32), ((0, N_PAD - n), (0, 0)))
  pad_idx = n + jnp.arange(E_PAD - e, dtype=jnp.int32) % (N_PAD - n)
  src = jnp.concatenate(
      [edge_index[0].astype(jnp.int32), pad_idx]).reshape(NCHUNK, CSZ)
  pad_dst = pad_idx
  dst = jnp.concatenate(
      [edge_index[1].astype(jnp.int32), pad_dst]).reshape(NCHUNK, CSZ)
  batch3 = jnp.concatenate(
      [batch.astype(jnp.int32),
       jnp.full((N_PAD - n,), PBUCK - 1, jnp.int32)]
  ).reshape(POOL_W, POOL_ROWS // CSZ, CSZ)

  ones8 = jnp.concatenate(
      [jnp.ones((CSZ, 1), jnp.float32), jnp.zeros((CSZ, 7), jnp.float32)], 1)
  zc8 = jnp.zeros((TROWS, 8), jnp.float32)

  dga, dgb = _sc_degree(dst, ones8, zc8)

  h1 = _tc_scale_matmul(xp, W1, dga, dgb)
  s1a, s1b = _sc_scatter(h1, src, dst)
  h2 = _tc_layer2(s1a, s1b, h1, dga, dgb, b1.reshape(1, DFEAT), W2)
  s2a, s2b = _sc_scatter(h2, src, dst)
  wfc8 = jnp.pad(Wfc, ((0, 0), (0, 7)))
  z8 = _tc_head(s2a, s2b, h2, dga, dgb, b2.reshape(1, DFEAT), wfc8)

  pa, pb = _sc_pool(z8, batch3, zc8)
  pool = pa + pb
  sums = pool[:NUM_G, 0]
  cnts = pool[:NUM_G, 1]
  return (sums / jnp.maximum(cnts, 1.0))[:, None] + bfc


# speedup vs baseline: 1.2007x; 1.2007x over previous
import functools

import jax
import jax.numpy as jnp
from jax import lax
from jax.experimental import pallas as pl
from jax.experimental.pallas import tpu as pltpu
from jax.experimental.pallas import tpu_sc as plsc

N_PAD = 10240
DFEAT = 128
NW = 32
CSZ = 128
NCHUNK = 2560
E_PAD = NCHUNK * CSZ
WCH = NCHUNK // NW
CPP = 40
DEG_CH = NCHUNK // NW
DEG_CPP = 40
TROWS = N_PAD // 16
DUMMY_DST = N_PAD - 1
PBUCK = 128
POOL_W = 16
POOL_ROWS = N_PAD // POOL_W
NUM_G = 64

_MESH = plsc.VectorSubcoreMesh(core_axis_name="c", subcore_axis_name="s")


def _sc_degree(dst3, ones8, zc8):
  @functools.partial(
      pl.kernel,
      out_type=(jax.ShapeDtypeStruct((N_PAD, 8), jnp.float32),
                jax.ShapeDtypeStruct((N_PAD, 8), jnp.float32)),
      mesh=_MESH,
      scratch_types=[
          pltpu.VMEM_SHARED((N_PAD, 8), jnp.float32),
          pltpu.VMEM((DEG_CPP, CSZ), jnp.int32),
          pltpu.VMEM((CSZ, 8), jnp.float32),
      ],
  )
  def k(dst_hbm, ones_hbm, zc_hbm, outa_hbm, outb_hbm, table, dst_v, ones_v):
    c = lax.axis_index("c")
    s = lax.axis_index("s")
    wid = c * 16 + s
    pltpu.sync_copy(zc_hbm, table.at[pl.ds(s * TROWS, TROWS)])
    pltpu.sync_copy(ones_hbm, ones_v)
    plsc.subcore_barrier()

    def body(j, carry):
      pltpu.sync_copy(ones_v, table.at[dst_v.at[j]], add=True)
      return carry

    for ph in range(DEG_CH // DEG_CPP):
      pltpu.sync_copy(dst_hbm.at[pl.ds(wid * DEG_CH + ph * DEG_CPP, DEG_CPP)],
                      dst_v)
      lax.fori_loop(0, DEG_CPP, body, 0)
    plsc.subcore_barrier()

    @pl.when(c == 0)
    def _():
      pltpu.sync_copy(table.at[pl.ds(s * TROWS, TROWS)],
                      outa_hbm.at[pl.ds(s * TROWS, TROWS)])

    @pl.when(c == 1)
    def _():
      pltpu.sync_copy(table.at[pl.ds(s * TROWS, TROWS)],
                      outb_hbm.at[pl.ds(s * TROWS, TROWS)])

  return k(dst3, ones8, zc8)


def _sc_scatter(m, src3, dst3):
  @functools.partial(
      pl.kernel,
      out_type=(jax.ShapeDtypeStruct((N_PAD, DFEAT), jnp.float32),
                jax.ShapeDtypeStruct((N_PAD, DFEAT), jnp.float32)),
      mesh=_MESH,
      scratch_types=[
          pltpu.VMEM_SHARED((N_PAD, DFEAT), jnp.float32),
          pltpu.VMEM((CPP, CSZ), jnp.int32),
          pltpu.VMEM((CPP, CSZ), jnp.int32),
          pltpu.VMEM((CSZ, DFEAT), jnp.float32),
          pltpu.VMEM((CSZ, DFEAT), jnp.float32),
          pltpu.SemaphoreType.DMA,
          pltpu.SemaphoreType.DMA,
      ],
  )
  def k(m_hbm, src_hbm, dst_hbm, outa_hbm, outb_hbm,
        acc, src_v, dst_v, rows0, rows1, sem0, sem1):
    c = lax.axis_index("c")
    s = lax.axis_index("s")
    wid = c * 16 + s

    def zbody(i, carry):
      rows0[i // 8, pl.ds((i % 8) * 16, 16)] = jnp.zeros((16,), jnp.float32)
      return carry

    lax.fori_loop(0, CSZ * 8, zbody, 0)
    for kblk in range(TROWS // CSZ):
      pltpu.sync_copy(rows0, acc.at[pl.ds(s * TROWS + kblk * CSZ, CSZ)])
    plsc.subcore_barrier()

    def body(i, carry):
      j0 = 2 * i
      j1 = j0 + 1
      pltpu.async_copy(m_hbm.at[src_v.at[j1]], rows1, sem1)
      pltpu.make_async_copy(m_hbm.at[src_v.at[j0]], rows0, sem0).wait()
      pltpu.sync_copy(rows0, acc.at[dst_v.at[j0]], add=True)

      @pl.when(j0 + 2 < CPP)
      def _():
        pltpu.async_copy(m_hbm.at[src_v.at[j0 + 2]], rows0, sem0)

      pltpu.make_async_copy(m_hbm.at[src_v.at[j1]], rows1, sem1).wait()
      pltpu.sync_copy(rows1, acc.at[dst_v.at[j1]], add=True)
      return carry

    def phase(ph, carry):
      base = wid * WCH + ph * CPP
      pltpu.sync_copy(src_hbm.at[pl.ds(base, CPP)], src_v)
      pltpu.sync_copy(dst_hbm.at[pl.ds(base, CPP)], dst_v)
      pltpu.async_copy(m_hbm.at[src_v.at[0]], rows0, sem0)
      lax.fori_loop(0, CPP // 2, body, 0)
      return carry

    lax.fori_loop(0, WCH // CPP, phase, 0)
    plsc.subcore_barrier()

    @pl.when(c == 0)
    def _():
      pltpu.sync_copy(acc.at[pl.ds(s * TROWS, TROWS)],
                      outa_hbm.at[pl.ds(s * TROWS, TROWS)])

    @pl.when(c == 1)
    def _():
      pltpu.sync_copy(acc.at[pl.ds(s * TROWS, TROWS)],
                      outb_hbm.at[pl.ds(s * TROWS, TROWS)])

  return k(m, src3, dst3)


def _sc_pool(z8, batch3, zc8):
  @functools.partial(
      pl.kernel,
      out_type=(jax.ShapeDtypeStruct((PBUCK, 8), jnp.float32),
                jax.ShapeDtypeStruct((PBUCK, 8), jnp.float32)),
      mesh=_MESH,
      scratch_types=[
          pltpu.VMEM_SHARED((PBUCK, 8), jnp.float32),
          pltpu.VMEM((POOL_ROWS // CSZ, CSZ), jnp.int32),
          pltpu.VMEM((POOL_ROWS, 8), jnp.float32),
      ],
  )
  def k(z_hbm, b_hbm, zc_hbm, outa_hbm, outb_hbm, table, b_v, z_v):
    c = lax.axis_index("c")
    s = lax.axis_index("s")

    @pl.when(s == 0)
    def _():
      pltpu.sync_copy(zc_hbm.at[pl.ds(0, PBUCK)], table)

    plsc.subcore_barrier()

    @pl.when(s < POOL_W // 2)
    def _():
      pw = c * (POOL_W // 2) + s
      pltpu.sync_copy(z_hbm.at[pl.ds(pw * POOL_ROWS, POOL_ROWS)], z_v)
      pltpu.sync_copy(b_hbm.at[pw], b_v)

      def body(j, carry):
        pltpu.sync_copy(z_v.at[pl.ds(j * CSZ, CSZ)],
                        table.at[b_v.at[j]], add=True)
        return carry

      lax.fori_loop(0, POOL_ROWS // CSZ, body, 0)

    plsc.subcore_barrier()

    @pl.when((s == 0) & (c == 0))
    def _():
      pltpu.sync_copy(table, outa_hbm)

    @pl.when((s == 0) & (c == 1))
    def _():
      pltpu.sync_copy(table, outb_hbm)

  return k(z8, batch3, zc8)


_BLK = 1024


def _tc_scale_matmul(x, w, dga, dgb):
  def body(x_ref, w_ref, da_ref, db_ref, o_ref):
    dinv = lax.rsqrt(da_ref[:, 0:1] + db_ref[:, 0:1] + 1.0)
    o_ref[...] = dinv * jnp.dot(x_ref[...], w_ref[...],
                                preferred_element_type=jnp.float32,
                                precision=lax.Precision.HIGHEST)

  return pl.pallas_call(
      body,
      grid=(N_PAD // _BLK,),
      in_specs=[
          pl.BlockSpec((_BLK, DFEAT), lambda i: (i, 0)),
          pl.BlockSpec((DFEAT, DFEAT), lambda i: (0, 0)),
          pl.BlockSpec((_BLK, 8), lambda i: (i, 0)),
          pl.BlockSpec((_BLK, 8), lambda i: (i, 0)),
      ],
      out_specs=pl.BlockSpec((_BLK, DFEAT), lambda i: (i, 0)),
      out_shape=jax.ShapeDtypeStruct((N_PAD, DFEAT), jnp.float32),
  )(x, w, dga, dgb)


def _tc_layer2(s1a, s1b, h1, dga, dgb, b1, w2):
  def body(sa_ref, sb_ref, h_ref, da_ref, db_ref, b_ref, w_ref, o_ref):
    dinv = lax.rsqrt(da_ref[:, 0:1] + db_ref[:, 0:1] + 1.0)
    t = dinv * (sa_ref[...] + sb_ref[...] + h_ref[...]) + b_ref[...]
    t = jnp.maximum(t, 0.0)
    o_ref[...] = dinv * jnp.dot(t, w_ref[...],
                                preferred_element_type=jnp.float32,
                                precision=lax.Precision.HIGHEST)

  return pl.pallas_call(
      body,
      grid=(N_PAD // _BLK,),
      in_specs=[
          pl.BlockSpec((_BLK, DFEAT), lambda i: (i, 0)),
          pl.BlockSpec((_BLK, DFEAT), lambda i: (i, 0)),
          pl.BlockSpec((_BLK, DFEAT), lambda i: (i, 0)),
          pl.BlockSpec((_BLK, 8), lambda i: (i, 0)),
          pl.BlockSpec((_BLK, 8), lambda i: (i, 0)),
          pl.BlockSpec((1, DFEAT), lambda i: (0, 0)),
          pl.BlockSpec((DFEAT, DFEAT), lambda i: (0, 0)),
      ],
      out_specs=pl.BlockSpec((_BLK, DFEAT), lambda i: (i, 0)),
      out_shape=jax.ShapeDtypeStruct((N_PAD, DFEAT), jnp.float32),
  )(s1a, s1b, h1, dga, dgb, b1, w2)


def _tc_head(s2a, s2b, h2, dga, dgb, b2, wfc8):
  def body(sa_ref, sb_ref, h_ref, da_ref, db_ref, b_ref, w_ref, o_ref):
    dinv = lax.rsqrt(da_ref[:, 0:1] + db_ref[:, 0:1] + 1.0)
    t = dinv * (sa_ref[...] + sb_ref[...] + h_ref[...]) + b_ref[...]
    z8 = jnp.dot(t, w_ref[...], preferred_element_type=jnp.float32,
                 precision=lax.Precision.HIGHEST)
    lane = lax.broadcasted_iota(jnp.int32, (_BLK, 8), 1)
    o_ref[...] = z8 + (lane == 1).astype(jnp.float32)

  return pl.pallas_call(
      body,
      grid=(N_PAD // _BLK,),
      in_specs=[
          pl.BlockSpec((_BLK, DFEAT), lambda i: (i, 0)),
          pl.BlockSpec((_BLK, DFEAT), lambda i: (i, 0)),
          pl.BlockSpec((_BLK, DFEAT), lambda i: (i, 0)),
          pl.BlockSpec((_BLK, 8), lambda i: (i, 0)),
          pl.BlockSpec((_BLK, 8), lambda i: (i, 0)),
          pl.BlockSpec((1, DFEAT), lambda i: (0, 0)),
          pl.BlockSpec((DFEAT, 8), lambda i: (0, 0)),
      ],
      out_specs=pl.BlockSpec((_BLK, 8), lambda i: (i, 0)),
      out_shape=jax.ShapeDtypeStruct((N_PAD, 8), jnp.float32),
  )(s2a, s2b, h2, dga, dgb, b2, wfc8)


def kernel(x, edge_index, batch, W1, b1, W2, b2, Wfc, bfc):
  n = x.shape[0]
  e = edge_index.shape[1]

  xp = jnp.pad(x.astype(jnp.float32), ((0, N_PAD - n), (0, 0)))
  pad_idx = n + jnp.arange(E_PAD - e, dtype=jnp.int32) % (N_PAD - n)
  src = jnp.concatenate(
      [edge_index[0].astype(jnp.int32), pad_idx]).reshape(NCHUNK, CSZ)
  pad_dst = pad_idx
  dst = jnp.concatenate(
      [edge_index[1].astype(jnp.int32), pad_dst]).reshape(NCHUNK, CSZ)
  batch3 = jnp.concatenate(
      [batch.astype(jnp.int32),
       jnp.full((N_PAD - n,), PBUCK - 1, jnp.int32)]
  ).reshape(POOL_W, POOL_ROWS // CSZ, CSZ)

  ones8 = jnp.concatenate(
      [jnp.ones((CSZ, 1), jnp.float32), jnp.zeros((CSZ, 7), jnp.float32)], 1)
  zc8 = jnp.zeros((TROWS, 8), jnp.float32)

  dga, dgb = _sc_degree(dst, ones8, zc8)

  h1 = _tc_scale_matmul(xp, W1, dga, dgb)
  s1a, s1b = _sc_scatter(h1, src, dst)
  h2 = _tc_layer2(s1a, s1b, h1, dga, dgb, b1.reshape(1, DFEAT), W2)
  s2a, s2b = _sc_scatter(h2, src, dst)
  wfc8 = jnp.pad(Wfc, ((0, 0), (0, 7)))
  z8 = _tc_head(s2a, s2b, h2, dga, dgb, b2.reshape(1, DFEAT), wfc8)

  pa, pb = _sc_pool(z8, batch3, zc8)
  pool = pa + pb
  sums = pool[:NUM_G, 0]
  cnts = pool[:NUM_G, 1]
  return (sums / jnp.maximum(cnts, 1.0))[:, None] + bfc


# speedup vs baseline: 1.2031x; 1.0020x over previous
import functools

import jax
import jax.numpy as jnp
from jax import lax
from jax.experimental import pallas as pl
from jax.experimental.pallas import tpu as pltpu
from jax.experimental.pallas import tpu_sc as plsc

N_PAD = 10240
DFEAT = 128
NW = 32
CSZ = 128
NCHUNK = 2560
E_PAD = NCHUNK * CSZ
WCH = NCHUNK // NW
CPP = 40
DEG_CH = NCHUNK // NW
DEG_CPP = 40
TROWS = N_PAD // 16
DUMMY_DST = N_PAD - 1
PBUCK = 128
POOL_W = 16
POOL_ROWS = N_PAD // POOL_W
NUM_G = 64

_MESH = plsc.VectorSubcoreMesh(core_axis_name="c", subcore_axis_name="s")


def _sc_degree(dst3, ones8, zc8):
  @functools.partial(
      pl.kernel,
      out_type=(jax.ShapeDtypeStruct((N_PAD, 8), jnp.float32),
                jax.ShapeDtypeStruct((N_PAD, 8), jnp.float32)),
      mesh=_MESH,
      scratch_types=[
          pltpu.VMEM_SHARED((N_PAD, 8), jnp.float32),
          pltpu.VMEM((DEG_CPP, CSZ), jnp.int32),
          pltpu.VMEM((CSZ, 8), jnp.float32),
      ],
  )
  def k(dst_hbm, ones_hbm, zc_hbm, outa_hbm, outb_hbm, table, dst_v, ones_v):
    c = lax.axis_index("c")
    s = lax.axis_index("s")
    wid = c * 16 + s
    pltpu.sync_copy(zc_hbm, table.at[pl.ds(s * TROWS, TROWS)])
    pltpu.sync_copy(ones_hbm, ones_v)
    plsc.subcore_barrier()

    def body(j, carry):
      pltpu.sync_copy(ones_v, table.at[dst_v.at[j]], add=True)
      return carry

    for ph in range(DEG_CH // DEG_CPP):
      pltpu.sync_copy(dst_hbm.at[pl.ds(wid * DEG_CH + ph * DEG_CPP, DEG_CPP)],
                      dst_v)
      lax.fori_loop(0, DEG_CPP, body, 0)
    plsc.subcore_barrier()

    @pl.when(c == 0)
    def _():
      pltpu.sync_copy(table.at[pl.ds(s * TROWS, TROWS)],
                      outa_hbm.at[pl.ds(s * TROWS, TROWS)])

    @pl.when(c == 1)
    def _():
      pltpu.sync_copy(table.at[pl.ds(s * TROWS, TROWS)],
                      outb_hbm.at[pl.ds(s * TROWS, TROWS)])

  return k(dst3, ones8, zc8)


def _sc_scatter(m, src3, dst3):
  @functools.partial(
      pl.kernel,
      out_type=(jax.ShapeDtypeStruct((N_PAD, DFEAT), jnp.float32),
                jax.ShapeDtypeStruct((N_PAD, DFEAT), jnp.float32)),
      mesh=_MESH,
      scratch_types=[
          pltpu.VMEM_SHARED((N_PAD, DFEAT), jnp.float32),
          pltpu.VMEM((CPP, CSZ), jnp.int32),
          pltpu.VMEM((CPP, CSZ), jnp.int32),
          pltpu.VMEM((CSZ, DFEAT), jnp.float32),
          pltpu.VMEM((CSZ, DFEAT), jnp.float32),
          pltpu.SemaphoreType.DMA,
          pltpu.SemaphoreType.DMA,
      ],
  )
  def k(m_hbm, src_hbm, dst_hbm, outa_hbm, outb_hbm,
        acc, src_v, dst_v, rows0, rows1, sem0, sem1):
    c = lax.axis_index("c")
    s = lax.axis_index("s")
    wid = c * 16 + s

    def zbody(i, carry):
      rows0[i // 8, pl.ds((i % 8) * 16, 16)] = jnp.zeros((16,), jnp.float32)
      return carry

    lax.fori_loop(0, CSZ * 8, zbody, 0)
    for kblk in range(TROWS // CSZ):
      pltpu.sync_copy(rows0, acc.at[pl.ds(s * TROWS + kblk * CSZ, CSZ)])
    plsc.subcore_barrier()

    def body(i, carry):
      j0 = 2 * i
      j1 = j0 + 1
      pltpu.async_copy(m_hbm.at[src_v.at[j1]], rows1, sem1)
      pltpu.make_async_copy(m_hbm.at[src_v.at[j0]], rows0, sem0).wait()
      pltpu.sync_copy(rows0, acc.at[dst_v.at[j0]], add=True)

      @pl.when(j0 + 2 < CPP)
      def _():
        pltpu.async_copy(m_hbm.at[src_v.at[j0 + 2]], rows0, sem0)

      pltpu.make_async_copy(m_hbm.at[src_v.at[j1]], rows1, sem1).wait()
      pltpu.sync_copy(rows1, acc.at[dst_v.at[j1]], add=True)
      return carry

    def phase(ph, carry):
      base = wid * WCH + ph * CPP
      pltpu.sync_copy(src_hbm.at[pl.ds(base, CPP)], src_v)
      pltpu.sync_copy(dst_hbm.at[pl.ds(base, CPP)], dst_v)
      pltpu.async_copy(m_hbm.at[src_v.at[0]], rows0, sem0)
      lax.fori_loop(0, CPP // 2, body, 0)
      return carry

    lax.fori_loop(0, WCH // CPP, phase, 0)
    plsc.subcore_barrier()

    @pl.when(c == 0)
    def _():
      pltpu.sync_copy(acc.at[pl.ds(s * TROWS, TROWS)],
                      outa_hbm.at[pl.ds(s * TROWS, TROWS)])

    @pl.when(c == 1)
    def _():
      pltpu.sync_copy(acc.at[pl.ds(s * TROWS, TROWS)],
                      outb_hbm.at[pl.ds(s * TROWS, TROWS)])

  return k(m, src3, dst3)


def _sc_pool(z8, batch3, zc8):
  @functools.partial(
      pl.kernel,
      out_type=(jax.ShapeDtypeStruct((PBUCK, 8), jnp.float32),
                jax.ShapeDtypeStruct((PBUCK, 8), jnp.float32)),
      mesh=_MESH,
      scratch_types=[
          pltpu.VMEM_SHARED((PBUCK, 8), jnp.float32),
          pltpu.VMEM((POOL_ROWS // CSZ, CSZ), jnp.int32),
          pltpu.VMEM((POOL_ROWS, 8), jnp.float32),
      ],
  )
  def k(z_hbm, b_hbm, zc_hbm, outa_hbm, outb_hbm, table, b_v, z_v):
    c = lax.axis_index("c")
    s = lax.axis_index("s")

    @pl.when(s == 0)
    def _():
      pltpu.sync_copy(zc_hbm.at[pl.ds(0, PBUCK)], table)

    plsc.subcore_barrier()

    @pl.when(s < POOL_W // 2)
    def _():
      pw = c * (POOL_W // 2) + s
      pltpu.sync_copy(z_hbm.at[pl.ds(pw * POOL_ROWS, POOL_ROWS)], z_v)
      pltpu.sync_copy(b_hbm.at[pw], b_v)

      def body(j, carry):
        pltpu.sync_copy(z_v.at[pl.ds(j * CSZ, CSZ)],
                        table.at[b_v.at[j]], add=True)
        return carry

      lax.fori_loop(0, POOL_ROWS // CSZ, body, 0)

    plsc.subcore_barrier()

    @pl.when((s == 0) & (c == 0))
    def _():
      pltpu.sync_copy(table, outa_hbm)

    @pl.when((s == 0) & (c == 1))
    def _():
      pltpu.sync_copy(table, outb_hbm)

  return k(z8, batch3, zc8)


_BLK = 1024


def _tc_matmul(x, w):
  def body(x_ref, w_ref, o_ref):
    o_ref[...] = jnp.dot(x_ref[...], w_ref[...],
                         preferred_element_type=jnp.float32,
                         precision=lax.Precision.HIGHEST)

  return pl.pallas_call(
      body,
      grid=(N_PAD // _BLK,),
      in_specs=[
          pl.BlockSpec((_BLK, DFEAT), lambda i: (i, 0)),
          pl.BlockSpec((DFEAT, DFEAT), lambda i: (0, 0)),
      ],
      out_specs=pl.BlockSpec((_BLK, DFEAT), lambda i: (i, 0)),
      out_shape=jax.ShapeDtypeStruct((N_PAD, DFEAT), jnp.float32),
  )(x, w)


def _tc_scale(p, dga, dgb):
  def body(p_ref, da_ref, db_ref, o_ref):
    dinv = lax.rsqrt(da_ref[:, 0:1] + db_ref[:, 0:1] + 1.0)
    o_ref[...] = dinv * p_ref[...]

  return pl.pallas_call(
      body,
      grid=(N_PAD // _BLK,),
      in_specs=[
          pl.BlockSpec((_BLK, DFEAT), lambda i: (i, 0)),
          pl.BlockSpec((_BLK, 8), lambda i: (i, 0)),
          pl.BlockSpec((_BLK, 8), lambda i: (i, 0)),
      ],
      out_specs=pl.BlockSpec((_BLK, DFEAT), lambda i: (i, 0)),
      out_shape=jax.ShapeDtypeStruct((N_PAD, DFEAT), jnp.float32),
  )(p, dga, dgb)


def _tc_layer2(s1a, s1b, h1, dga, dgb, b1, w2):
  def body(sa_ref, sb_ref, h_ref, da_ref, db_ref, b_ref, w_ref, o_ref):
    dinv = lax.rsqrt(da_ref[:, 0:1] + db_ref[:, 0:1] + 1.0)
    t = dinv * (sa_ref[...] + sb_ref[...] + h_ref[...]) + b_ref[...]
    t = jnp.maximum(t, 0.0)
    o_ref[...] = dinv * jnp.dot(t, w_ref[...],
                                preferred_element_type=jnp.float32,
                                precision=lax.Precision.HIGHEST)

  return pl.pallas_call(
      body,
      grid=(N_PAD // _BLK,),
      in_specs=[
          pl.BlockSpec((_BLK, DFEAT), lambda i: (i, 0)),
          pl.BlockSpec((_BLK, DFEAT), lambda i: (i, 0)),
          pl.BlockSpec((_BLK, DFEAT), lambda i: (i, 0)),
          pl.BlockSpec((_BLK, 8), lambda i: (i, 0)),
          pl.BlockSpec((_BLK, 8), lambda i: (i, 0)),
          pl.BlockSpec((1, DFEAT), lambda i: (0, 0)),
          pl.BlockSpec((DFEAT, DFEAT), lambda i: (0, 0)),
      ],
      out_specs=pl.BlockSpec((_BLK, DFEAT), lambda i: (i, 0)),
      out_shape=jax.ShapeDtypeStruct((N_PAD, DFEAT), jnp.float32),
  )(s1a, s1b, h1, dga, dgb, b1, w2)


def _tc_head(s2a, s2b, h2, dga, dgb, b2, wfc8):
  def body(sa_ref, sb_ref, h_ref, da_ref, db_ref, b_ref, w_ref, o_ref):
    dinv = lax.rsqrt(da_ref[:, 0:1] + db_ref[:, 0:1] + 1.0)
    t = dinv * (sa_ref[...] + sb_ref[...] + h_ref[...]) + b_ref[...]
    z8 = jnp.dot(t, w_ref[...], preferred_element_type=jnp.float32,
                 precision=lax.Precision.HIGHEST)
    lane = lax.broadcasted_iota(jnp.int32, (_BLK, 8), 1)
    o_ref[...] = z8 + (lane == 1).astype(jnp.float32)

  return pl.pallas_call(
      body,
      grid=(N_PAD // _BLK,),
      in_specs=[
          pl.BlockSpec((_BLK, DFEAT), lambda i: (i, 0)),
          pl.BlockSpec((_BLK, DFEAT), lambda i: (i, 0)),
          pl.BlockSpec((_BLK, DFEAT), lambda i: (i, 0)),
          pl.BlockSpec((_BLK, 8), lambda i: (i, 0)),
          pl.BlockSpec((_BLK, 8), lambda i: (i, 0)),
          pl.BlockSpec((1, DFEAT), lambda i: (0, 0)),
          pl.BlockSpec((DFEAT, 8), lambda i: (0, 0)),
      ],
      out_specs=pl.BlockSpec((_BLK, 8), lambda i: (i, 0)),
      out_shape=jax.ShapeDtypeStruct((N_PAD, 8), jnp.float32),
  )(s2a, s2b, h2, dga, dgb, b2, wfc8)


def kernel(x, edge_index, batch, W1, b1, W2, b2, Wfc, bfc):
  n = x.shape[0]
  e = edge_index.shape[1]

  xp = jnp.pad(x.astype(jnp.float32), ((0, N_PAD - n), (0, 0)))
  pad_idx = n + jnp.arange(E_PAD - e, dtype=jnp.int32) % (N_PAD - n)
  src = jnp.concatenate(
      [edge_index[0].astype(jnp.int32), pad_idx]).reshape(NCHUNK, CSZ)
  pad_dst = pad_idx
  dst = jnp.concatenate(
      [edge_index[1].astype(jnp.int32), pad_dst]).reshape(NCHUNK, CSZ)
  batch3 = jnp.concatenate(
      [batch.astype(jnp.int32),
       jnp.full((N_PAD - n,), PBUCK - 1, jnp.int32)]
  ).reshape(POOL_W, POOL_ROWS // CSZ, CSZ)

  ones8 = jnp.concatenate(
      [jnp.ones((CSZ, 1), jnp.float32), jnp.zeros((CSZ, 7), jnp.float32)], 1)
  zc8 = jnp.zeros((TROWS, 8), jnp.float32)

  p1 = _tc_matmul(xp, W1)
  dga, dgb = _sc_degree(dst, ones8, zc8)
  h1 = _tc_scale(p1, dga, dgb)
  s1a, s1b = _sc_scatter(h1, src, dst)
  h2 = _tc_layer2(s1a, s1b, h1, dga, dgb, b1.reshape(1, DFEAT), W2)
  s2a, s2b = _sc_scatter(h2, src, dst)
  wfc8 = jnp.pad(Wfc, ((0, 0), (0, 7)))
  z8 = _tc_head(s2a, s2b, h2, dga, dgb, b2.reshape(1, DFEAT), wfc8)

  pa, pb = _sc_pool(z8, batch3, zc8)
  pool = pa + pb
  sums = pool[:NUM_G, 0]
  cnts = pool[:NUM_G, 1]
  return (sums / jnp.maximum(cnts, 1.0))[:, None] + bfc


# speedup vs baseline: 1.2051x; 1.0016x over previous
import functools

import jax
import jax.numpy as jnp
from jax import lax
from jax.experimental import pallas as pl
from jax.experimental.pallas import tpu as pltpu
from jax.experimental.pallas import tpu_sc as plsc

N_PAD = 10240
DFEAT = 128
NW = 32
CSZ = 128
NCHUNK = 2560
E_PAD = NCHUNK * CSZ
WCH = NCHUNK // NW
CPP = 40
DEG_CH = NCHUNK // NW
DEG_CPP = 40
TROWS = N_PAD // 16
PBUCK = 128
POOL_W = 16
POOL_ROWS = N_PAD // POOL_W
NUM_G = 64

_MESH = plsc.VectorSubcoreMesh(core_axis_name="c", subcore_axis_name="s")


def _sc_degree(dst3, ones8, zc8):
  @functools.partial(
      pl.kernel,
      out_type=(jax.ShapeDtypeStruct((N_PAD, 8), jnp.float32),
                jax.ShapeDtypeStruct((N_PAD, 8), jnp.float32)),
      mesh=_MESH,
      scratch_types=[
          pltpu.VMEM_SHARED((N_PAD, 8), jnp.float32),
          pltpu.VMEM((DEG_CPP, CSZ), jnp.int32),
          pltpu.VMEM((CSZ, 8), jnp.float32),
      ],
  )
  def k(dst_hbm, ones_hbm, zc_hbm, outa_hbm, outb_hbm, table, dst_v, ones_v):
    c = lax.axis_index("c")
    s = lax.axis_index("s")
    wid = c * 16 + s
    pltpu.sync_copy(zc_hbm, table.at[pl.ds(s * TROWS, TROWS)])
    pltpu.sync_copy(ones_hbm, ones_v)
    plsc.subcore_barrier()

    def body(j, carry):
      pltpu.sync_copy(ones_v, table.at[dst_v.at[j]], add=True)
      return carry

    for ph in range(DEG_CH // DEG_CPP):
      pltpu.sync_copy(dst_hbm.at[pl.ds(wid * DEG_CH + ph * DEG_CPP, DEG_CPP)],
                      dst_v)
      lax.fori_loop(0, DEG_CPP, body, 0)
    plsc.subcore_barrier()

    @pl.when(c == 0)
    def _():
      pltpu.sync_copy(table.at[pl.ds(s * TROWS, TROWS)],
                      outa_hbm.at[pl.ds(s * TROWS, TROWS)])

    @pl.when(c == 1)
    def _():
      pltpu.sync_copy(table.at[pl.ds(s * TROWS, TROWS)],
                      outb_hbm.at[pl.ds(s * TROWS, TROWS)])

  return k(dst3, ones8, zc8)


def _sc_scatter(m, src3, dst3):
  @functools.partial(
      pl.kernel,
      out_type=(jax.ShapeDtypeStruct((N_PAD, DFEAT), jnp.float32),
                jax.ShapeDtypeStruct((N_PAD, DFEAT), jnp.float32)),
      mesh=_MESH,
      scratch_types=[
          pltpu.VMEM_SHARED((N_PAD, DFEAT), jnp.float32),
          pltpu.VMEM((CPP, CSZ), jnp.int32),
          pltpu.VMEM((CPP, CSZ), jnp.int32),
          pltpu.VMEM((CSZ, DFEAT), jnp.float32),
          pltpu.VMEM((CSZ, DFEAT), jnp.float32),
          pltpu.SemaphoreType.DMA,
          pltpu.SemaphoreType.DMA,
      ],
  )
  def k(m_hbm, src_hbm, dst_hbm, outa_hbm, outb_hbm,
        acc, src_v, dst_v, rows0, rows1, sem0, sem1):
    c = lax.axis_index("c")
    s = lax.axis_index("s")
    wid = c * 16 + s

    def zbody(i, carry):
      rows0[i // 8, pl.ds((i % 8) * 16, 16)] = jnp.zeros((16,), jnp.float32)
      return carry

    lax.fori_loop(0, CSZ * 8, zbody, 0)
    for kblk in range(TROWS // CSZ):
      pltpu.sync_copy(rows0, acc.at[pl.ds(s * TROWS + kblk * CSZ, CSZ)])
    plsc.subcore_barrier()

    def body(i, carry):
      j0 = 2 * i
      j1 = j0 + 1
      pltpu.async_copy(m_hbm.at[src_v.at[j1]], rows1, sem1)
      pltpu.make_async_copy(m_hbm.at[src_v.at[j0]], rows0, sem0).wait()
      pltpu.sync_copy(rows0, acc.at[dst_v.at[j0]], add=True)

      @pl.when(j0 + 2 < CPP)
      def _():
        pltpu.async_copy(m_hbm.at[src_v.at[j0 + 2]], rows0, sem0)

      pltpu.make_async_copy(m_hbm.at[src_v.at[j1]], rows1, sem1).wait()
      pltpu.sync_copy(rows1, acc.at[dst_v.at[j1]], add=True)
      return carry

    def phase(ph, carry):
      base = wid * WCH + ph * CPP
      pltpu.sync_copy(src_hbm.at[pl.ds(base, CPP)], src_v)
      pltpu.sync_copy(dst_hbm.at[pl.ds(base, CPP)], dst_v)
      pltpu.async_copy(m_hbm.at[src_v.at[0]], rows0, sem0)
      lax.fori_loop(0, CPP // 2, body, 0)
      return carry

    lax.fori_loop(0, WCH // CPP, phase, 0)
    plsc.subcore_barrier()

    @pl.when(c == 0)
    def _():
      pltpu.sync_copy(acc.at[pl.ds(s * TROWS, TROWS)],
                      outa_hbm.at[pl.ds(s * TROWS, TROWS)])

    @pl.when(c == 1)
    def _():
      pltpu.sync_copy(acc.at[pl.ds(s * TROWS, TROWS)],
                      outb_hbm.at[pl.ds(s * TROWS, TROWS)])

  return k(m, src3, dst3)


def _sc_pool(z8, batch3, zc8):
  @functools.partial(
      pl.kernel,
      out_type=(jax.ShapeDtypeStruct((PBUCK, 8), jnp.float32),
                jax.ShapeDtypeStruct((PBUCK, 8), jnp.float32)),
      mesh=_MESH,
      scratch_types=[
          pltpu.VMEM_SHARED((PBUCK, 8), jnp.float32),
          pltpu.VMEM((POOL_ROWS // CSZ, CSZ), jnp.int32),
          pltpu.VMEM((POOL_ROWS, 8), jnp.float32),
      ],
  )
  def k(z_hbm, b_hbm, zc_hbm, outa_hbm, outb_hbm, table, b_v, z_v):
    c = lax.axis_index("c")
    s = lax.axis_index("s")

    @pl.when(s == 0)
    def _():
      pltpu.sync_copy(zc_hbm.at[pl.ds(0, PBUCK)], table)

    plsc.subcore_barrier()

    @pl.when(s < POOL_W // 2)
    def _():
      pw = c * (POOL_W // 2) + s
      pltpu.sync_copy(z_hbm.at[pl.ds(pw * POOL_ROWS, POOL_ROWS)], z_v)
      pltpu.sync_copy(b_hbm.at[pw], b_v)

      def body(j, carry):
        pltpu.sync_copy(z_v.at[pl.ds(j * CSZ, CSZ)],
                        table.at[b_v.at[j]], add=True)
        return carry

      lax.fori_loop(0, POOL_ROWS // CSZ, body, 0)

    plsc.subcore_barrier()

    @pl.when((s == 0) & (c == 0))
    def _():
      pltpu.sync_copy(table, outa_hbm)

    @pl.when((s == 0) & (c == 1))
    def _():
      pltpu.sync_copy(table, outb_hbm)

  return k(z8, batch3, zc8)


_BLK = 1024


def _tc_matmul(x, w):
  def body(x_ref, w_ref, o_ref):
    o_ref[...] = jnp.dot(x_ref[...], w_ref[...],
                         preferred_element_type=jnp.float32,
                         precision=lax.Precision.HIGHEST)

  return pl.pallas_call(
      body,
      grid=(N_PAD // _BLK,),
      in_specs=[
          pl.BlockSpec((_BLK, DFEAT), lambda i: (i, 0)),
          pl.BlockSpec((DFEAT, DFEAT), lambda i: (0, 0)),
      ],
      out_specs=pl.BlockSpec((_BLK, DFEAT), lambda i: (i, 0)),
      out_shape=jax.ShapeDtypeStruct((N_PAD, DFEAT), jnp.float32),
  )(x, w)


def _tc_scale(p, dga, dgb):
  def body(p_ref, da_ref, db_ref, o_ref):
    dinv = lax.rsqrt(da_ref[:, 0:1] + db_ref[:, 0:1] + 1.0)
    o_ref[...] = dinv * p_ref[...]

  return pl.pallas_call(
      body,
      grid=(N_PAD // _BLK,),
      in_specs=[
          pl.BlockSpec((_BLK, DFEAT), lambda i: (i, 0)),
          pl.BlockSpec((_BLK, 8), lambda i: (i, 0)),
          pl.BlockSpec((_BLK, 8), lambda i: (i, 0)),
      ],
      out_specs=pl.BlockSpec((_BLK, DFEAT), lambda i: (i, 0)),
      out_shape=jax.ShapeDtypeStruct((N_PAD, DFEAT), jnp.float32),
  )(p, dga, dgb)


def _tc_layer2(s1a, s1b, h1, dga, dgb, b1, w2):
  def body(sa_ref, sb_ref, h_ref, da_ref, db_ref, b_ref, w_ref, o_ref):
    dinv = lax.rsqrt(da_ref[:, 0:1] + db_ref[:, 0:1] + 1.0)
    t = dinv * (sa_ref[...] + sb_ref[...] + h_ref[...]) + b_ref[...]
    t = jnp.maximum(t, 0.0)
    o_ref[...] = dinv * jnp.dot(t, w_ref[...],
                                preferred_element_type=jnp.float32,
                                precision=lax.Precision.HIGHEST)

  return pl.pallas_call(
      body,
      grid=(N_PAD // _BLK,),
      in_specs=[
          pl.BlockSpec((_BLK, DFEAT), lambda i: (i, 0)),
          pl.BlockSpec((_BLK, DFEAT), lambda i: (i, 0)),
          pl.BlockSpec((_BLK, DFEAT), lambda i: (i, 0)),
          pl.BlockSpec((_BLK, 8), lambda i: (i, 0)),
          pl.BlockSpec((_BLK, 8), lambda i: (i, 0)),
          pl.BlockSpec((1, DFEAT), lambda i: (0, 0)),
          pl.BlockSpec((DFEAT, DFEAT), lambda i: (0, 0)),
      ],
      out_specs=pl.BlockSpec((_BLK, DFEAT), lambda i: (i, 0)),
      out_shape=jax.ShapeDtypeStruct((N_PAD, DFEAT), jnp.float32),
  )(s1a, s1b, h1, dga, dgb, b1, w2)


def _tc_head(s2a, s2b, h2, dga, dgb, b2, wfc8):
  def body(sa_ref, sb_ref, h_ref, da_ref, db_ref, b_ref, w_ref, o_ref):
    dinv = lax.rsqrt(da_ref[:, 0:1] + db_ref[:, 0:1] + 1.0)
    t = dinv * (sa_ref[...] + sb_ref[...] + h_ref[...]) + b_ref[...]
    z8 = jnp.dot(t, w_ref[...], preferred_element_type=jnp.float32,
                 precision=lax.Precision.HIGHEST)
    lane = lax.broadcasted_iota(jnp.int32, (_BLK, 8), 1)
    o_ref[...] = z8 + (lane == 1).astype(jnp.float32)

  return pl.pallas_call(
      body,
      grid=(N_PAD // _BLK,),
      in_specs=[
          pl.BlockSpec((_BLK, DFEAT), lambda i: (i, 0)),
          pl.BlockSpec((_BLK, DFEAT), lambda i: (i, 0)),
          pl.BlockSpec((_BLK, DFEAT), lambda i: (i, 0)),
          pl.BlockSpec((_BLK, 8), lambda i: (i, 0)),
          pl.BlockSpec((_BLK, 8), lambda i: (i, 0)),
          pl.BlockSpec((1, DFEAT), lambda i: (0, 0)),
          pl.BlockSpec((DFEAT, 8), lambda i: (0, 0)),
      ],
      out_specs=pl.BlockSpec((_BLK, 8), lambda i: (i, 0)),
      out_shape=jax.ShapeDtypeStruct((N_PAD, 8), jnp.float32),
  )(s2a, s2b, h2, dga, dgb, b2, wfc8)


def kernel(x, edge_index, batch, W1, b1, W2, b2, Wfc, bfc):
  n = x.shape[0]
  e = edge_index.shape[1]

  xp = jnp.pad(x.astype(jnp.float32), ((0, N_PAD - n), (0, 0)))
  pad_idx = n + jnp.arange(E_PAD - e, dtype=jnp.int32) % (N_PAD - n)
  src = jnp.concatenate(
      [edge_index[0].astype(jnp.int32), pad_idx]).reshape(NCHUNK, CSZ)
  pad_dst = pad_idx
  dst = jnp.concatenate(
      [edge_index[1].astype(jnp.int32), pad_dst]).reshape(NCHUNK, CSZ)
  batch3 = jnp.concatenate(
      [batch.astype(jnp.int32),
       jnp.full((N_PAD - n,), PBUCK - 1, jnp.int32)]
  ).reshape(POOL_W, POOL_ROWS // CSZ, CSZ)

  ones8 = jnp.concatenate(
      [jnp.ones((CSZ, 1), jnp.float32), jnp.zeros((CSZ, 7), jnp.float32)], 1)
  zc8 = jnp.zeros((TROWS, 8), jnp.float32)

  p1 = _tc_matmul(xp, W1)
  dga, dgb = _sc_degree(dst, ones8, zc8)
  h1 = _tc_scale(p1, dga, dgb)
  s1a, s1b = _sc_scatter(h1, src, dst)
  h2 = _tc_layer2(s1a, s1b, h1, dga, dgb, b1.reshape(1, DFEAT), W2)
  s2a, s2b = _sc_scatter(h2, src, dst)
  wfc8 = jnp.pad(Wfc, ((0, 0), (0, 7)))
  z8 = _tc_head(s2a, s2b, h2, dga, dgb, b2.reshape(1, DFEAT), wfc8)

  pa, pb = _sc_pool(z8, batch3, zc8)
  pool = pa + pb
  sums = pool[:NUM_G, 0]
  cnts = pool[:NUM_G, 1]
  return (sums / jnp.maximum(cnts, 1.0))[:, None] + bfc
